# Initial kernel scaffold; baseline (speedup 1.0000x reference)
import jax
import jax.numpy as jnp
from jax.experimental import pallas as pl


def kernel(x, Wq, keys, Wv, core0, core1, qn_scale, kn_scale, values, shuffle_index):
    raise NotImplementedError("write your pallas kernel here")


# speedup vs baseline: 5.5321x; 5.5321x over previous
import dataclasses
import functools

import jax
import jax.numpy as jnp
from jax import lax
from jax.experimental import pallas as pl
from jax.experimental.pallas import tpu as pltpu
from jax.experimental.pallas import tpu_sc as plsc

HIDDEN = 4096
KDIM = 128
KEY_NUM = 128
VALUE_NUM = KEY_NUM * KEY_NUM
VDIM = 1024
KNN = 32
HEAD = 2
RANK = 2
MHEAD = 2
NGROUP = HEAD * MHEAD
EPS = 1e-5
NEG = -1e30


TB = 256


def _ln_last(v, scale):
    m = jnp.mean(v, axis=-1, keepdims=True)
    c = v - m
    var = jnp.mean(c * c, axis=-1, keepdims=True)
    return c * lax.rsqrt(var + EPS) * scale


def _topk_stage1(s_r0, s_r1):
    cur0 = s_r0 + s_r1
    iota_k = lax.broadcasted_iota(jnp.int32, (TB, KEY_NUM), 1).astype(
        jnp.float32)
    iota_o = lax.broadcasted_iota(jnp.int32, (TB, KNN), 1)
    z = jnp.zeros((TB, KNN), jnp.float32)

    def body(j, carry):
        cur, i_f, g0, g1 = carry
        mval = jnp.max(cur, axis=-1, keepdims=True)
        eq = cur == mval
        idxf = jnp.min(jnp.where(eq, iota_k, 1e9), axis=-1, keepdims=True)
        onehot = iota_k == idxf
        cur = jnp.where(onehot, NEG, cur)
        oh = onehot.astype(jnp.float32)
        v0 = jnp.sum(oh * s_r0, axis=-1, keepdims=True)
        v1 = jnp.sum(oh * s_r1, axis=-1, keepdims=True)
        colj = iota_o == j
        return (cur,
                jnp.where(colj, idxf, i_f),
                jnp.where(colj, v0, g0),
                jnp.where(colj, v1, g1))

    _, i_f, g0, g1 = lax.fori_loop(0, KNN, body, (cur0, z, z, z))
    return i_f, g0, g1


def _stage_a_body(x_ref, wqt_ref, kmat_ref, qn_ref, kn_ref, c_ref,
                  w_ref, vi_ref):
    q = jnp.dot(x_ref[...].astype(jnp.bfloat16),
                wqt_ref[...].astype(jnp.bfloat16),
                preferred_element_type=jnp.float32)
    q1 = _ln_last(q[:, :KDIM], qn_ref[...])
    q2 = _ln_last(q[:, KDIM:], qn_ref[...])

    kmat = kmat_ref[...]
    km = jnp.mean(kmat, axis=1, keepdims=True)
    kc = kmat - km
    kv = jnp.mean(kc * kc, axis=1, keepdims=True)
    knorm = (kc * lax.rsqrt(kv + EPS) * kn_ref[...]).astype(jnp.bfloat16)
    s1 = jnp.dot(q1.astype(jnp.bfloat16), knorm[0],
                 preferred_element_type=jnp.float32)
    s2 = jnp.dot(q2.astype(jnp.bfloat16), knorm[1],
                 preferred_element_type=jnp.float32)

    NN = KNN * KNN
    row_i = lax.broadcasted_iota(jnp.int32, (KNN, NN), 0)
    col_i = lax.broadcasted_iota(jnp.int32, (KNN, NN), 1)
    e_a = (row_i == (col_i // KNN)).astype(jnp.bfloat16)
    e_b = (row_i == (col_i % KNN)).astype(jnp.bfloat16)

    iota_nn = lax.broadcasted_iota(jnp.int32, (TB, NN), 1).astype(jnp.float32)
    iota_kn = lax.broadcasted_iota(jnp.int32, (TB, KNN), 1).astype(jnp.float32)
    iota_ko = lax.broadcasted_iota(jnp.int32, (TB, KNN), 1)

    for h in range(HEAD):
        c0 = KEY_NUM * h
        i1f, g1a, g1b = _topk_stage1(s1[:, c0:c0 + KEY_NUM],
                                     s1[:, c0 + HEAD * KEY_NUM:
                                         c0 + HEAD * KEY_NUM + KEY_NUM])
        i2f, g2a, g2b = _topk_stage1(s2[:, c0:c0 + KEY_NUM],
                                     s2[:, c0 + HEAD * KEY_NUM:
                                         c0 + HEAD * KEY_NUM + KEY_NUM])
        g1ab = g1a.astype(jnp.bfloat16).astype(jnp.float32)
        g1bb = g1b.astype(jnp.bfloat16).astype(jnp.float32)
        g2a_e = jnp.dot(g2a.astype(jnp.bfloat16), e_b,
                        preferred_element_type=jnp.float32)
        g2b_e = jnp.dot(g2b.astype(jnp.bfloat16), e_b,
                        preferred_element_type=jnp.float32)
        for m in range(MHEAD):
            cm = [[lax.convert_element_type(
                       lax.convert_element_type(
                           c_ref[((m * HEAD + h) * RANK + r) * RANK + s],
                           jnp.bfloat16), jnp.float32)
                   for s in range(RANK)] for r in range(RANK)]
            u0 = g1ab * cm[0][0] + g1bb * cm[1][0]
            u1 = g1ab * cm[0][1] + g1bb * cm[1][1]
            u0e = jnp.dot(u0.astype(jnp.bfloat16), e_a,
                          preferred_element_type=jnp.float32)
            u1e = jnp.dot(u1.astype(jnp.bfloat16), e_a,
                          preferred_element_type=jnp.float32)
            cur0 = u0e * g2a_e + u1e * g2b_e

            g = h * MHEAD + m
            z = jnp.zeros((TB, KNN), jnp.float32)

            def body2(j, carry, i1f=i1f, i2f=i2f):
                cur, tvals, viacc = carry
                mval = jnp.max(cur, axis=-1, keepdims=True)
                eq = cur == mval
                idxf = jnp.min(jnp.where(eq, iota_nn, 1e9),
                               axis=-1, keepdims=True)
                onehot = iota_nn == idxf
                cur = jnp.where(onehot, NEG, cur)
                a_f = jnp.floor(idxf * (1.0 / KNN))
                b_f = idxf - a_f * KNN
                oh_a = (iota_kn == a_f).astype(jnp.float32)
                oh_b = (iota_kn == b_f).astype(jnp.float32)
                sel1 = jnp.sum(oh_a * i1f, axis=-1, keepdims=True)
                sel2 = jnp.sum(oh_b * i2f, axis=-1, keepdims=True)
                colj = iota_ko == j
                return (cur,
                        jnp.where(colj, mval, tvals),
                        jnp.where(colj, sel1 * KEY_NUM + sel2, viacc))

            _, tvals, viacc = lax.fori_loop(0, KNN, body2, (cur0, z, z))
            ex = jnp.exp(tvals - jnp.max(tvals, axis=-1, keepdims=True))
            w = ex / jnp.sum(ex, axis=-1, keepdims=True) * (1.0 / NGROUP)
            w_ref[:, g * KNN:(g + 1) * KNN] = w
            vi_ref[:, g * KNN:(g + 1) * KNN] = viacc.astype(jnp.int32)


def _stage_a(x, wqt, kmat, qn, kn, cflat):
    t = x.shape[0]
    grid = t // TB
    return pl.pallas_call(
        _stage_a_body,
        grid=(grid,),
        in_specs=[
            pl.BlockSpec((TB, HIDDEN), lambda i: (i, 0)),
            pl.BlockSpec((HIDDEN, 2 * KDIM), lambda i: (0, 0)),
            pl.BlockSpec((2, KDIM, RANK * HEAD * KEY_NUM), lambda i: (0, 0, 0)),
            pl.BlockSpec((1, KDIM), lambda i: (0, 0)),
            pl.BlockSpec((1, KDIM, 1), lambda i: (0, 0, 0)),
            pl.BlockSpec(memory_space=pltpu.SMEM),
        ],
        out_specs=[
            pl.BlockSpec((TB, NGROUP * KNN), lambda i: (i, 0)),
            pl.BlockSpec((TB, NGROUP * KNN), lambda i: (i, 0)),
        ],
        out_shape=[
            jax.ShapeDtypeStruct((t, NGROUP * KNN), jnp.float32),
            jax.ShapeDtypeStruct((t, NGROUP * KNN), jnp.int32),
        ],
        compiler_params=pltpu.CompilerParams(
            dimension_semantics=("arbitrary",)),
    )(x, wqt, kmat, qn, kn, cflat)


NC, NS, NLANE = 2, 16, 16
NW = NC * NS
CH = 16
NBUF = 2


def _sc_gather(values, shuffle, vi_flat):
    b = vi_flat.shape[0]
    bpw = b // NW
    mesh = plsc.VectorSubcoreMesh(core_axis_name="c", subcore_axis_name="s")
    cp = pltpu.CompilerParams()
    if "needs_layout_passes" in pltpu.CompilerParams.__dataclass_fields__:
        cp = dataclasses.replace(cp, needs_layout_passes=False)

    @functools.partial(
        pl.kernel,
        mesh=mesh,
        compiler_params=cp,
        out_type=jax.ShapeDtypeStruct((b, VDIM), jnp.float32),
        scratch_types=[
            pltpu.VMEM((VALUE_NUM,), jnp.int32),
            pltpu.VMEM((bpw,), jnp.int32),
            pltpu.VMEM((bpw,), jnp.int32),
            pltpu.VMEM((NBUF, CH, VDIM), jnp.float32),
            pltpu.SemaphoreType.DMA,
            pltpu.SemaphoreType.DMA,
            pltpu.SemaphoreType.DMA,
            pltpu.SemaphoreType.DMA,
        ],
    )
    def k(values_hbm, shuf_hbm, vi_hbm, out_hbm,
          shuf_v, vi_v, vidx_v, rows_v, gsem0, gsem1, wsem0, wsem1):
        wid = lax.axis_index("s") * NC + lax.axis_index("c")
        base = wid * bpw
        pltpu.sync_copy(shuf_hbm, shuf_v)
        pltpu.sync_copy(vi_hbm.at[pl.ds(base, bpw)], vi_v)

        @pl.loop(0, bpw, step=NLANE)
        def _resolve(i):
            idx = vi_v[pl.ds(i, NLANE)]
            vidx_v[pl.ds(i, NLANE)] = plsc.load_gather(shuf_v, [idx])

        gsems = [gsem0, gsem1]
        wsems = [wsem0, wsem1]
        nch = bpw // CH

        for sl in range(NBUF):
            pltpu.async_copy(values_hbm.at[vidx_v.at[pl.ds(sl * CH, CH)]],
                             rows_v.at[sl], gsems[sl])

        @pl.loop(0, nch, step=NBUF)
        def _chunks(c):
            for sl in range(NBUF):
                cur = c + sl
                pltpu.make_async_copy(
                    values_hbm.at[vidx_v.at[pl.ds(cur * CH, CH)]],
                    rows_v.at[sl], gsems[sl]).wait()
                pltpu.async_copy(
                    rows_v.at[sl],
                    out_hbm.at[pl.ds(base + cur * CH, CH)], wsems[sl])
                pltpu.make_async_copy(
                    rows_v.at[sl],
                    out_hbm.at[pl.ds(base + cur * CH, CH)], wsems[sl]).wait()

                @pl.when(cur + NBUF < nch)
                def _():
                    pltpu.async_copy(
                        values_hbm.at[vidx_v.at[pl.ds((cur + NBUF) * CH, CH)]],
                        rows_v.at[sl], gsems[sl])

    return k(values, shuffle, vi_flat)


TC_B = 16


def _stage_c_body(rows_ref, w_ref, wvt_ref, out_ref):
    rows3 = rows_ref[...].astype(jnp.bfloat16).astype(jnp.float32).reshape(
        TC_B, NGROUP * KNN, VDIM)
    w3 = w_ref[...].astype(jnp.bfloat16).astype(jnp.float32).reshape(
        TC_B, NGROUP * KNN, 1)
    agg = jnp.sum(rows3 * w3, axis=1)
    out_ref[...] = jnp.dot(agg.astype(jnp.bfloat16), wvt_ref[...],
                           preferred_element_type=jnp.float32)


def _stage_c(rows, w, wvt):
    t = w.shape[0]
    grid = t // TC_B
    nr = NGROUP * KNN
    return pl.pallas_call(
        _stage_c_body,
        grid=(grid,),
        in_specs=[
            pl.BlockSpec((TC_B * nr, VDIM), lambda i: (i, 0)),
            pl.BlockSpec((TC_B, nr), lambda i: (i, 0)),
            pl.BlockSpec((VDIM, HIDDEN), lambda i: (0, 0)),
        ],
        out_specs=pl.BlockSpec((TC_B, HIDDEN), lambda i: (i, 0)),
        out_shape=jax.ShapeDtypeStruct((t, HIDDEN), jnp.float32),
        compiler_params=pltpu.CompilerParams(
            dimension_semantics=("arbitrary",)),
    )(rows, w.astype(jnp.float32), wvt)


def kernel(x, Wq, keys, Wv, core0, core1, qn_scale, kn_scale, values,
           shuffle_index):
    t = x.shape[0]
    wqt = Wq.T
    kmat = jnp.transpose(keys, (1, 3, 4, 0, 2)).reshape(
        2, KDIM, RANK * HEAD * KEY_NUM)
    qn = qn_scale.reshape(1, KDIM)
    kn = kn_scale.reshape(1, KDIM, 1)
    cflat = jnp.stack([core0, core1], axis=0).reshape(-1)

    w, vi = _stage_a(x, wqt, kmat, qn, kn, cflat)
    rows = _sc_gather(values, shuffle_index, vi.reshape(-1))
    wvt = Wv.T.astype(jnp.bfloat16)
    return _stage_c(rows, w, wvt)


# speedup vs baseline: 5.5323x; 1.0000x over previous
import dataclasses
import functools

import jax
import jax.numpy as jnp
from jax import lax
from jax.experimental import pallas as pl
from jax.experimental.pallas import tpu as pltpu
from jax.experimental.pallas import tpu_sc as plsc

HIDDEN = 4096
KDIM = 128
KEY_NUM = 128
VALUE_NUM = KEY_NUM * KEY_NUM
VDIM = 1024
KNN = 32
HEAD = 2
RANK = 2
MHEAD = 2
NGROUP = HEAD * MHEAD
EPS = 1e-5
NEG = -1e30


TB = 256


def _ln_last(v, scale):
    m = jnp.mean(v, axis=-1, keepdims=True)
    c = v - m
    var = jnp.mean(c * c, axis=-1, keepdims=True)
    return c * lax.rsqrt(var + EPS) * scale


def _topk_stage1(s_r0, s_r1):
    cur0 = s_r0 + s_r1
    iota_k = lax.broadcasted_iota(jnp.int32, (TB, KEY_NUM), 1).astype(
        jnp.float32)
    iota_o = lax.broadcasted_iota(jnp.int32, (TB, KNN), 1)
    z = jnp.zeros((TB, KNN), jnp.float32)

    def body(j, carry):
        cur, i_f, g0, g1 = carry
        mval = jnp.max(cur, axis=-1, keepdims=True)
        eq = cur == mval
        idxf = jnp.min(jnp.where(eq, iota_k, 1e9), axis=-1, keepdims=True)
        onehot = iota_k == idxf
        cur = jnp.where(onehot, NEG, cur)
        oh = onehot.astype(jnp.float32)
        v0 = jnp.sum(oh * s_r0, axis=-1, keepdims=True)
        v1 = jnp.sum(oh * s_r1, axis=-1, keepdims=True)
        colj = iota_o == j
        return (cur,
                jnp.where(colj, idxf, i_f),
                jnp.where(colj, v0, g0),
                jnp.where(colj, v1, g1))

    _, i_f, g0, g1 = lax.fori_loop(0, KNN, body, (cur0, z, z, z))
    return i_f, g0, g1


def _stage_a_body(x_ref, wqt_ref, kmat_ref, qn_ref, kn_ref, c_ref,
                  w_ref, vi_ref):
    q = jnp.dot(x_ref[...].astype(jnp.bfloat16),
                wqt_ref[...].astype(jnp.bfloat16),
                preferred_element_type=jnp.float32)
    q1 = _ln_last(q[:, :KDIM], qn_ref[...])
    q2 = _ln_last(q[:, KDIM:], qn_ref[...])

    kmat = kmat_ref[...]
    km = jnp.mean(kmat, axis=1, keepdims=True)
    kc = kmat - km
    kv = jnp.mean(kc * kc, axis=1, keepdims=True)
    knorm = (kc * lax.rsqrt(kv + EPS) * kn_ref[...]).astype(jnp.bfloat16)
    s1 = jnp.dot(q1.astype(jnp.bfloat16), knorm[0],
                 preferred_element_type=jnp.float32)
    s2 = jnp.dot(q2.astype(jnp.bfloat16), knorm[1],
                 preferred_element_type=jnp.float32)

    NN = KNN * KNN
    row_i = lax.broadcasted_iota(jnp.int32, (KNN, NN), 0)
    col_i = lax.broadcasted_iota(jnp.int32, (KNN, NN), 1)
    e_a = (row_i == (col_i // KNN)).astype(jnp.bfloat16)
    e_b = (row_i == (col_i % KNN)).astype(jnp.bfloat16)

    iota_nn = lax.broadcasted_iota(jnp.int32, (TB, NN), 1).astype(jnp.float32)
    iota_kn = lax.broadcasted_iota(jnp.int32, (TB, KNN), 1).astype(jnp.float32)
    iota_ko = lax.broadcasted_iota(jnp.int32, (TB, KNN), 1)

    for h in range(HEAD):
        c0 = KEY_NUM * h
        i1f, g1a, g1b = _topk_stage1(s1[:, c0:c0 + KEY_NUM],
                                     s1[:, c0 + HEAD * KEY_NUM:
                                         c0 + HEAD * KEY_NUM + KEY_NUM])
        i2f, g2a, g2b = _topk_stage1(s2[:, c0:c0 + KEY_NUM],
                                     s2[:, c0 + HEAD * KEY_NUM:
                                         c0 + HEAD * KEY_NUM + KEY_NUM])
        g1ab = g1a.astype(jnp.bfloat16).astype(jnp.float32)
        g1bb = g1b.astype(jnp.bfloat16).astype(jnp.float32)
        g2a_e = jnp.dot(g2a.astype(jnp.bfloat16), e_b,
                        preferred_element_type=jnp.float32)
        g2b_e = jnp.dot(g2b.astype(jnp.bfloat16), e_b,
                        preferred_element_type=jnp.float32)
        for m in range(MHEAD):
            cm = [[lax.convert_element_type(
                       lax.convert_element_type(
                           c_ref[((m * HEAD + h) * RANK + r) * RANK + s],
                           jnp.bfloat16), jnp.float32)
                   for s in range(RANK)] for r in range(RANK)]
            u0 = g1ab * cm[0][0] + g1bb * cm[1][0]
            u1 = g1ab * cm[0][1] + g1bb * cm[1][1]
            u0e = jnp.dot(u0.astype(jnp.bfloat16), e_a,
                          preferred_element_type=jnp.float32)
            u1e = jnp.dot(u1.astype(jnp.bfloat16), e_a,
                          preferred_element_type=jnp.float32)
            cur0 = u0e * g2a_e + u1e * g2b_e

            g = h * MHEAD + m
            z = jnp.zeros((TB, KNN), jnp.float32)

            def body2(j, carry, i1f=i1f, i2f=i2f):
                cur, tvals, viacc = carry
                mval = jnp.max(cur, axis=-1, keepdims=True)
                eq = cur == mval
                idxf = jnp.min(jnp.where(eq, iota_nn, 1e9),
                               axis=-1, keepdims=True)
                onehot = iota_nn == idxf
                cur = jnp.where(onehot, NEG, cur)
                a_f = jnp.floor(idxf * (1.0 / KNN))
                b_f = idxf - a_f * KNN
                oh_a = (iota_kn == a_f).astype(jnp.float32)
                oh_b = (iota_kn == b_f).astype(jnp.float32)
                sel1 = jnp.sum(oh_a * i1f, axis=-1, keepdims=True)
                sel2 = jnp.sum(oh_b * i2f, axis=-1, keepdims=True)
                colj = iota_ko == j
                return (cur,
                        jnp.where(colj, mval, tvals),
                        jnp.where(colj, sel1 * KEY_NUM + sel2, viacc))

            _, tvals, viacc = lax.fori_loop(0, KNN, body2, (cur0, z, z))
            ex = jnp.exp(tvals - jnp.max(tvals, axis=-1, keepdims=True))
            w = ex / jnp.sum(ex, axis=-1, keepdims=True) * (1.0 / NGROUP)
            w_ref[:, g * KNN:(g + 1) * KNN] = w
            vi_ref[:, g * KNN:(g + 1) * KNN] = viacc.astype(jnp.int32)


def _stage_a(x, wqt, kmat, qn, kn, cflat):
    t = x.shape[0]
    grid = t // TB
    return pl.pallas_call(
        _stage_a_body,
        grid=(grid,),
        in_specs=[
            pl.BlockSpec((TB, HIDDEN), lambda i: (i, 0)),
            pl.BlockSpec((HIDDEN, 2 * KDIM), lambda i: (0, 0)),
            pl.BlockSpec((2, KDIM, RANK * HEAD * KEY_NUM), lambda i: (0, 0, 0)),
            pl.BlockSpec((1, KDIM), lambda i: (0, 0)),
            pl.BlockSpec((1, KDIM, 1), lambda i: (0, 0, 0)),
            pl.BlockSpec(memory_space=pltpu.SMEM),
        ],
        out_specs=[
            pl.BlockSpec((TB, NGROUP * KNN), lambda i: (i, 0)),
            pl.BlockSpec((TB, NGROUP * KNN), lambda i: (i, 0)),
        ],
        out_shape=[
            jax.ShapeDtypeStruct((t, NGROUP * KNN), jnp.float32),
            jax.ShapeDtypeStruct((t, NGROUP * KNN), jnp.int32),
        ],
        compiler_params=pltpu.CompilerParams(
            dimension_semantics=("parallel",)),
    )(x, wqt, kmat, qn, kn, cflat)


NC, NS, NLANE = 2, 16, 16
NW = NC * NS
CH = 32
NBUF = 2


def _sc_gather(values, shuffle, vi_flat):
    b = vi_flat.shape[0]
    bpw = b // NW
    mesh = plsc.VectorSubcoreMesh(core_axis_name="c", subcore_axis_name="s")
    cp = pltpu.CompilerParams()
    if "needs_layout_passes" in pltpu.CompilerParams.__dataclass_fields__:
        cp = dataclasses.replace(cp, needs_layout_passes=False)

    @functools.partial(
        pl.kernel,
        mesh=mesh,
        compiler_params=cp,
        out_type=jax.ShapeDtypeStruct((b, VDIM), jnp.float32),
        scratch_types=[
            pltpu.VMEM((VALUE_NUM,), jnp.int32),
            pltpu.VMEM((bpw,), jnp.int32),
            pltpu.VMEM((bpw,), jnp.int32),
            pltpu.VMEM((NBUF, CH, VDIM), jnp.float32),
            pltpu.SemaphoreType.DMA,
            pltpu.SemaphoreType.DMA,
            pltpu.SemaphoreType.DMA,
            pltpu.SemaphoreType.DMA,
        ],
    )
    def k(values_hbm, shuf_hbm, vi_hbm, out_hbm,
          shuf_v, vi_v, vidx_v, rows_v, gsem0, gsem1, wsem0, wsem1):
        wid = lax.axis_index("s") * NC + lax.axis_index("c")
        base = wid * bpw
        pltpu.sync_copy(shuf_hbm, shuf_v)
        pltpu.sync_copy(vi_hbm.at[pl.ds(base, bpw)], vi_v)

        @pl.loop(0, bpw, step=NLANE)
        def _resolve(i):
            idx = vi_v[pl.ds(i, NLANE)]
            vidx_v[pl.ds(i, NLANE)] = plsc.load_gather(shuf_v, [idx])

        gsems = [gsem0, gsem1]
        wsems = [wsem0, wsem1]
        nch = bpw // CH

        for sl in range(NBUF):
            pltpu.async_copy(values_hbm.at[vidx_v.at[pl.ds(sl * CH, CH)]],
                             rows_v.at[sl], gsems[sl])

        @pl.loop(0, nch, step=NBUF)
        def _chunks(c):
            for sl in range(NBUF):
                cur = c + sl
                pltpu.make_async_copy(
                    values_hbm.at[vidx_v.at[pl.ds(cur * CH, CH)]],
                    rows_v.at[sl], gsems[sl]).wait()
                pltpu.async_copy(
                    rows_v.at[sl],
                    out_hbm.at[pl.ds(base + cur * CH, CH)], wsems[sl])
                prev = cur - 1
                psl = (sl + NBUF - 1) % NBUF

                @pl.when((prev >= 0) & (prev + NBUF < nch))
                def _():
                    pltpu.make_async_copy(
                        rows_v.at[psl],
                        out_hbm.at[pl.ds(base + prev * CH, CH)],
                        wsems[psl]).wait()
                    pltpu.async_copy(
                        values_hbm.at[vidx_v.at[pl.ds((prev + NBUF) * CH, CH)]],
                        rows_v.at[psl], gsems[psl])

        for sl in range(NBUF):
            cur = nch - NBUF + sl
            pltpu.make_async_copy(
                rows_v.at[(cur % NBUF)],
                out_hbm.at[pl.ds(base + cur * CH, CH)],
                wsems[(cur % NBUF)]).wait()

    return k(values, shuffle, vi_flat)


TC_B = 16


def _stage_c_body(rows_ref, w_ref, wvt_ref, out_ref):
    rows3 = rows_ref[...].astype(jnp.bfloat16).astype(jnp.float32).reshape(
        TC_B, NGROUP * KNN, VDIM)
    w3 = w_ref[...].astype(jnp.bfloat16).astype(jnp.float32).reshape(
        TC_B, NGROUP * KNN, 1)
    agg = jnp.sum(rows3 * w3, axis=1)
    out_ref[...] = jnp.dot(agg.astype(jnp.bfloat16), wvt_ref[...],
                           preferred_element_type=jnp.float32)


def _stage_c(rows, w, wvt):
    t = w.shape[0]
    grid = t // TC_B
    nr = NGROUP * KNN
    return pl.pallas_call(
        _stage_c_body,
        grid=(grid,),
        in_specs=[
            pl.BlockSpec((TC_B * nr, VDIM), lambda i: (i, 0)),
            pl.BlockSpec((TC_B, nr), lambda i: (i, 0)),
            pl.BlockSpec((VDIM, HIDDEN), lambda i: (0, 0)),
        ],
        out_specs=pl.BlockSpec((TC_B, HIDDEN), lambda i: (i, 0)),
        out_shape=jax.ShapeDtypeStruct((t, HIDDEN), jnp.float32),
        compiler_params=pltpu.CompilerParams(
            dimension_semantics=("parallel",)),
    )(rows, w.astype(jnp.float32), wvt)


def kernel(x, Wq, keys, Wv, core0, core1, qn_scale, kn_scale, values,
           shuffle_index):
    t = x.shape[0]
    wqt = Wq.T
    kmat = jnp.transpose(keys, (1, 3, 4, 0, 2)).reshape(
        2, KDIM, RANK * HEAD * KEY_NUM)
    qn = qn_scale.reshape(1, KDIM)
    kn = kn_scale.reshape(1, KDIM, 1)
    cflat = jnp.stack([core0, core1], axis=0).reshape(-1)

    w, vi = _stage_a(x, wqt, kmat, qn, kn, cflat)
    rows = _sc_gather(values, shuffle_index, vi.reshape(-1))
    wvt = Wv.T.astype(jnp.bfloat16)
    return _stage_c(rows, w, wvt)


# speedup vs baseline: 7.6072x; 1.3750x over previous
import dataclasses
import functools

import jax
import jax.numpy as jnp
from jax import lax
from jax.experimental import pallas as pl
from jax.experimental.pallas import tpu as pltpu
from jax.experimental.pallas import tpu_sc as plsc

HIDDEN = 4096
KDIM = 128
KEY_NUM = 128
VALUE_NUM = KEY_NUM * KEY_NUM
VDIM = 1024
KNN = 32
HEAD = 2
RANK = 2
MHEAD = 2
NGROUP = HEAD * MHEAD
EPS = 1e-5
NEG = -1e30


TB = 256


def _ln_last(v, scale):
    m = jnp.mean(v, axis=-1, keepdims=True)
    c = v - m
    var = jnp.mean(c * c, axis=-1, keepdims=True)
    return c * lax.rsqrt(var + EPS) * scale


def _topk_stage1(s_r0, s_r1):
    cur0 = s_r0 + s_r1
    iota_k = lax.broadcasted_iota(jnp.int32, (TB, KEY_NUM), 1).astype(
        jnp.float32)
    iota_o = lax.broadcasted_iota(jnp.int32, (TB, KNN), 1)
    z = jnp.zeros((TB, KNN), jnp.float32)

    def body(j, carry):
        cur, i_f, g0, g1 = carry
        mval = jnp.max(cur, axis=-1, keepdims=True)
        eq = cur == mval
        idxf = jnp.min(jnp.where(eq, iota_k, 1e9), axis=-1, keepdims=True)
        onehot = iota_k == idxf
        cur = jnp.where(onehot, NEG, cur)
        oh = onehot.astype(jnp.float32)
        v0 = jnp.sum(oh * s_r0, axis=-1, keepdims=True)
        v1 = jnp.sum(oh * s_r1, axis=-1, keepdims=True)
        colj = iota_o == j
        return (cur,
                jnp.where(colj, idxf, i_f),
                jnp.where(colj, v0, g0),
                jnp.where(colj, v1, g1))

    _, i_f, g0, g1 = lax.fori_loop(0, KNN, body, (cur0, z, z, z))
    return i_f, g0, g1


def _stage_a_body(x_ref, wqt_ref, kmat_ref, qn_ref, kn_ref, c_ref,
                  w_ref, vi_ref):
    q = jnp.dot(x_ref[...].astype(jnp.bfloat16),
                wqt_ref[...].astype(jnp.bfloat16),
                preferred_element_type=jnp.float32)
    q1 = _ln_last(q[:, :KDIM], qn_ref[...])
    q2 = _ln_last(q[:, KDIM:], qn_ref[...])

    kmat = kmat_ref[...]
    km = jnp.mean(kmat, axis=1, keepdims=True)
    kc = kmat - km
    kv = jnp.mean(kc * kc, axis=1, keepdims=True)
    knorm = (kc * lax.rsqrt(kv + EPS) * kn_ref[...]).astype(jnp.bfloat16)
    s1 = jnp.dot(q1.astype(jnp.bfloat16), knorm[0],
                 preferred_element_type=jnp.float32)
    s2 = jnp.dot(q2.astype(jnp.bfloat16), knorm[1],
                 preferred_element_type=jnp.float32)

    NN = KNN * KNN
    row_i = lax.broadcasted_iota(jnp.int32, (KNN, NN), 0)
    col_i = lax.broadcasted_iota(jnp.int32, (KNN, NN), 1)
    e_a = (row_i == (col_i // KNN)).astype(jnp.bfloat16)
    e_b = (row_i == (col_i % KNN)).astype(jnp.bfloat16)

    iota_nn = lax.broadcasted_iota(jnp.int32, (TB, NN), 1).astype(jnp.float32)
    iota_kn = lax.broadcasted_iota(jnp.int32, (TB, KNN), 1).astype(jnp.float32)
    iota_ko = lax.broadcasted_iota(jnp.int32, (TB, KNN), 1)

    for h in range(HEAD):
        c0 = KEY_NUM * h
        i1f, g1a, g1b = _topk_stage1(s1[:, c0:c0 + KEY_NUM],
                                     s1[:, c0 + HEAD * KEY_NUM:
                                         c0 + HEAD * KEY_NUM + KEY_NUM])
        i2f, g2a, g2b = _topk_stage1(s2[:, c0:c0 + KEY_NUM],
                                     s2[:, c0 + HEAD * KEY_NUM:
                                         c0 + HEAD * KEY_NUM + KEY_NUM])
        g1ab = g1a.astype(jnp.bfloat16).astype(jnp.float32)
        g1bb = g1b.astype(jnp.bfloat16).astype(jnp.float32)
        g2a_e = jnp.dot(g2a.astype(jnp.bfloat16), e_b,
                        preferred_element_type=jnp.float32)
        g2b_e = jnp.dot(g2b.astype(jnp.bfloat16), e_b,
                        preferred_element_type=jnp.float32)
        for m in range(MHEAD):
            cm = [[lax.convert_element_type(
                       lax.convert_element_type(
                           c_ref[((m * HEAD + h) * RANK + r) * RANK + s],
                           jnp.bfloat16), jnp.float32)
                   for s in range(RANK)] for r in range(RANK)]
            u0 = g1ab * cm[0][0] + g1bb * cm[1][0]
            u1 = g1ab * cm[0][1] + g1bb * cm[1][1]
            u0e = jnp.dot(u0.astype(jnp.bfloat16), e_a,
                          preferred_element_type=jnp.float32)
            u1e = jnp.dot(u1.astype(jnp.bfloat16), e_a,
                          preferred_element_type=jnp.float32)
            cur0 = u0e * g2a_e + u1e * g2b_e

            g = h * MHEAD + m
            z = jnp.zeros((TB, KNN), jnp.float32)

            def body2(j, carry, i1f=i1f, i2f=i2f):
                cur, tvals, viacc = carry
                mval = jnp.max(cur, axis=-1, keepdims=True)
                eq = cur == mval
                idxf = jnp.min(jnp.where(eq, iota_nn, 1e9),
                               axis=-1, keepdims=True)
                onehot = iota_nn == idxf
                cur = jnp.where(onehot, NEG, cur)
                a_f = jnp.floor(idxf * (1.0 / KNN))
                b_f = idxf - a_f * KNN
                oh_a = (iota_kn == a_f).astype(jnp.float32)
                oh_b = (iota_kn == b_f).astype(jnp.float32)
                sel1 = jnp.sum(oh_a * i1f, axis=-1, keepdims=True)
                sel2 = jnp.sum(oh_b * i2f, axis=-1, keepdims=True)
                colj = iota_ko == j
                return (cur,
                        jnp.where(colj, mval, tvals),
                        jnp.where(colj, sel1 * KEY_NUM + sel2, viacc))

            _, tvals, viacc = lax.fori_loop(0, KNN, body2, (cur0, z, z))
            ex = jnp.exp(tvals - jnp.max(tvals, axis=-1, keepdims=True))
            w = ex / jnp.sum(ex, axis=-1, keepdims=True) * (1.0 / NGROUP)
            w_ref[:, g * KNN:(g + 1) * KNN] = w
            vi_ref[:, g * KNN:(g + 1) * KNN] = viacc.astype(jnp.int32)


def _stage_a(x, wqt, kmat, qn, kn, cflat):
    t = x.shape[0]
    grid = t // TB
    return pl.pallas_call(
        _stage_a_body,
        grid=(grid,),
        in_specs=[
            pl.BlockSpec((TB, HIDDEN), lambda i: (i, 0)),
            pl.BlockSpec((HIDDEN, 2 * KDIM), lambda i: (0, 0)),
            pl.BlockSpec((2, KDIM, RANK * HEAD * KEY_NUM), lambda i: (0, 0, 0)),
            pl.BlockSpec((1, KDIM), lambda i: (0, 0)),
            pl.BlockSpec((1, KDIM, 1), lambda i: (0, 0, 0)),
            pl.BlockSpec(memory_space=pltpu.SMEM),
        ],
        out_specs=[
            pl.BlockSpec((TB, NGROUP * KNN), lambda i: (i, 0)),
            pl.BlockSpec((TB, NGROUP * KNN), lambda i: (i, 0)),
        ],
        out_shape=[
            jax.ShapeDtypeStruct((t, NGROUP * KNN), jnp.float32),
            jax.ShapeDtypeStruct((t, NGROUP * KNN), jnp.int32),
        ],
        compiler_params=pltpu.CompilerParams(
            dimension_semantics=("parallel",)),
    )(x, wqt, kmat, qn, kn, cflat)


NC, NS, NLANE = 2, 16, 16
NW = NC * NS
CH = 32
NBUF = 2


def _round_vals_body(v_ref, o_ref):
    o_ref[...] = v_ref[...].astype(jnp.bfloat16).astype(jnp.float32)


def _round_vals(values):
    rb = 512
    return pl.pallas_call(
        _round_vals_body,
        grid=(VALUE_NUM // rb,),
        in_specs=[pl.BlockSpec((rb, VDIM), lambda i: (i, 0))],
        out_specs=pl.BlockSpec((rb, VDIM), lambda i: (i, 0)),
        out_shape=jax.ShapeDtypeStruct((VALUE_NUM, VDIM), jnp.float32),
        compiler_params=pltpu.CompilerParams(
            dimension_semantics=("parallel",)),
    )(values)


def _bf16_round16(x):
    u = lax.bitcast_convert_type(x, jnp.uint32)
    r = (u + jnp.uint32(0x7FFF) + ((u >> jnp.uint32(16)) & jnp.uint32(1))) \
        & jnp.uint32(0xFFFF0000)
    return lax.bitcast_convert_type(r, jnp.float32)


def _sc_gather_combine(values, shuffle, vi_flat, w_flat):
    b = vi_flat.shape[0]
    bpw = b // NW
    tpw = bpw // (NGROUP * KNN)
    t_total = b // (NGROUP * KNN)
    nch = bpw // CH
    cpt = (NGROUP * KNN) // CH
    mesh = plsc.VectorSubcoreMesh(core_axis_name="c", subcore_axis_name="s")
    cp = pltpu.CompilerParams()
    if "needs_layout_passes" in pltpu.CompilerParams.__dataclass_fields__:
        cp = dataclasses.replace(cp, needs_layout_passes=False)

    @functools.partial(
        pl.kernel,
        mesh=mesh,
        compiler_params=cp,
        out_type=jax.ShapeDtypeStruct((t_total, VDIM), jnp.float32),
        scratch_types=[
            pltpu.VMEM((VALUE_NUM,), jnp.int32),
            pltpu.VMEM((bpw,), jnp.int32),
            pltpu.VMEM((bpw,), jnp.int32),
            pltpu.VMEM((bpw,), jnp.float32),
            pltpu.VMEM((NBUF * CH, VDIM), jnp.float32),
            pltpu.VMEM((1, VDIM), jnp.float32),
            pltpu.SemaphoreType.DMA,
            pltpu.SemaphoreType.DMA,
        ],
    )
    def k(values_hbm, shuf_hbm, vi_hbm, w_hbm, out_hbm,
          shuf_v, vi_v, vidx_v, w_v, rows_v, acc_v, gsem0, gsem1):
        wid = lax.axis_index("s") * NC + lax.axis_index("c")
        base = wid * bpw
        tok0 = wid * tpw
        pltpu.sync_copy(shuf_hbm, shuf_v)
        pltpu.sync_copy(vi_hbm.at[pl.ds(base, bpw)], vi_v)
        pltpu.sync_copy(w_hbm.at[pl.ds(base, bpw)], w_v)

        @pl.loop(0, bpw, step=NLANE)
        def _resolve(i):
            idx = vi_v[pl.ds(i, NLANE)]
            vidx_v[pl.ds(i, NLANE)] = plsc.load_gather(shuf_v, [idx])

        gsems = [gsem0, gsem1]
        for sl in range(NBUF):
            pltpu.async_copy(values_hbm.at[vidx_v.at[pl.ds(sl * CH, CH)]],
                             rows_v.at[pl.ds(sl * CH, CH)], gsems[sl])

        @pl.loop(0, nch, step=NBUF)
        def _chunks(c):
            for sl in range(NBUF):
                cur = c + sl
                tok = cur // cpt
                phase = cur - tok * cpt
                pltpu.make_async_copy(
                    values_hbm.at[vidx_v.at[pl.ds(cur * CH, CH)]],
                    rows_v.at[pl.ds(sl * CH, CH)], gsems[sl]).wait()
                wvecs = [_bf16_round16(plsc.load_gather(
                            w_v, [jnp.full((NLANE,), cur * CH + j,
                                           jnp.int32)]))
                         for j in range(CH)]
                keep = jnp.full((NLANE,), phase > 0)

                @pl.loop(0, VDIM, step=NLANE)
                def _acc(l):
                    sl16 = pl.ds(l, NLANE)
                    a0 = jnp.where(keep, acc_v[0, sl16],
                                   jnp.zeros((NLANE,), jnp.float32))
                    a1 = jnp.zeros((NLANE,), jnp.float32)
                    for j in range(CH):
                        term = wvecs[j] * rows_v[sl * CH + j, sl16]
                        if j % 2 == 0:
                            a0 = a0 + term
                        else:
                            a1 = a1 + term
                    acc_v[0, sl16] = a0 + a1

                @pl.when(phase == cpt - 1)
                def _():
                    pltpu.sync_copy(acc_v, out_hbm.at[pl.ds(tok0 + tok, 1)])

                @pl.when(cur + NBUF < nch)
                def _():
                    pltpu.async_copy(
                        values_hbm.at[vidx_v.at[pl.ds((cur + NBUF) * CH, CH)]],
                        rows_v.at[pl.ds(sl * CH, CH)], gsems[sl])

    return k(values, shuffle, vi_flat, w_flat)


def _sc_gather(values, shuffle, vi_flat):
    b = vi_flat.shape[0]
    bpw = b // NW
    mesh = plsc.VectorSubcoreMesh(core_axis_name="c", subcore_axis_name="s")
    cp = pltpu.CompilerParams()
    if "needs_layout_passes" in pltpu.CompilerParams.__dataclass_fields__:
        cp = dataclasses.replace(cp, needs_layout_passes=False)

    @functools.partial(
        pl.kernel,
        mesh=mesh,
        compiler_params=cp,
        out_type=jax.ShapeDtypeStruct((b, VDIM), jnp.float32),
        scratch_types=[
            pltpu.VMEM((VALUE_NUM,), jnp.int32),
            pltpu.VMEM((bpw,), jnp.int32),
            pltpu.VMEM((bpw,), jnp.int32),
            pltpu.VMEM((NBUF, CH, VDIM), jnp.float32),
            pltpu.SemaphoreType.DMA,
            pltpu.SemaphoreType.DMA,
            pltpu.SemaphoreType.DMA,
            pltpu.SemaphoreType.DMA,
        ],
    )
    def k(values_hbm, shuf_hbm, vi_hbm, out_hbm,
          shuf_v, vi_v, vidx_v, rows_v, gsem0, gsem1, wsem0, wsem1):
        wid = lax.axis_index("s") * NC + lax.axis_index("c")
        base = wid * bpw
        pltpu.sync_copy(shuf_hbm, shuf_v)
        pltpu.sync_copy(vi_hbm.at[pl.ds(base, bpw)], vi_v)

        @pl.loop(0, bpw, step=NLANE)
        def _resolve(i):
            idx = vi_v[pl.ds(i, NLANE)]
            vidx_v[pl.ds(i, NLANE)] = plsc.load_gather(shuf_v, [idx])

        gsems = [gsem0, gsem1]
        wsems = [wsem0, wsem1]
        nch = bpw // CH

        for sl in range(NBUF):
            pltpu.async_copy(values_hbm.at[vidx_v.at[pl.ds(sl * CH, CH)]],
                             rows_v.at[sl], gsems[sl])

        @pl.loop(0, nch, step=NBUF)
        def _chunks(c):
            for sl in range(NBUF):
                cur = c + sl
                pltpu.make_async_copy(
                    values_hbm.at[vidx_v.at[pl.ds(cur * CH, CH)]],
                    rows_v.at[sl], gsems[sl]).wait()
                pltpu.async_copy(
                    rows_v.at[sl],
                    out_hbm.at[pl.ds(base + cur * CH, CH)], wsems[sl])
                prev = cur - 1
                psl = (sl + NBUF - 1) % NBUF

                @pl.when((prev >= 0) & (prev + NBUF < nch))
                def _():
                    pltpu.make_async_copy(
                        rows_v.at[psl],
                        out_hbm.at[pl.ds(base + prev * CH, CH)],
                        wsems[psl]).wait()
                    pltpu.async_copy(
                        values_hbm.at[vidx_v.at[pl.ds((prev + NBUF) * CH, CH)]],
                        rows_v.at[psl], gsems[psl])

        for sl in range(NBUF):
            cur = nch - NBUF + sl
            pltpu.make_async_copy(
                rows_v.at[(cur % NBUF)],
                out_hbm.at[pl.ds(base + cur * CH, CH)],
                wsems[(cur % NBUF)]).wait()

    return k(values, shuffle, vi_flat)


TC_B = 16


def _stage_c_body(rows_ref, w_ref, wvt_ref, out_ref):
    rows3 = rows_ref[...].astype(jnp.bfloat16).astype(jnp.float32).reshape(
        TC_B, NGROUP * KNN, VDIM)
    w3 = w_ref[...].astype(jnp.bfloat16).astype(jnp.float32).reshape(
        TC_B, NGROUP * KNN, 1)
    agg = jnp.sum(rows3 * w3, axis=1)
    out_ref[...] = jnp.dot(agg.astype(jnp.bfloat16), wvt_ref[...],
                           preferred_element_type=jnp.float32)


def _stage_c(rows, w, wvt):
    t = w.shape[0]
    grid = t // TC_B
    nr = NGROUP * KNN
    return pl.pallas_call(
        _stage_c_body,
        grid=(grid,),
        in_specs=[
            pl.BlockSpec((TC_B * nr, VDIM), lambda i: (i, 0)),
            pl.BlockSpec((TC_B, nr), lambda i: (i, 0)),
            pl.BlockSpec((VDIM, HIDDEN), lambda i: (0, 0)),
        ],
        out_specs=pl.BlockSpec((TC_B, HIDDEN), lambda i: (i, 0)),
        out_shape=jax.ShapeDtypeStruct((t, HIDDEN), jnp.float32),
        compiler_params=pltpu.CompilerParams(
            dimension_semantics=("parallel",)),
    )(rows, w.astype(jnp.float32), wvt)


def _stage_proj_body(agg_ref, wvt_ref, out_ref):
    out_ref[...] = jnp.dot(agg_ref[...].astype(jnp.bfloat16), wvt_ref[...],
                           preferred_element_type=jnp.float32)


def _stage_proj(agg, wvt):
    t = agg.shape[0]
    tb = 256
    return pl.pallas_call(
        _stage_proj_body,
        grid=(t // tb,),
        in_specs=[
            pl.BlockSpec((tb, VDIM), lambda i: (i, 0)),
            pl.BlockSpec((VDIM, HIDDEN), lambda i: (0, 0)),
        ],
        out_specs=pl.BlockSpec((tb, HIDDEN), lambda i: (i, 0)),
        out_shape=jax.ShapeDtypeStruct((t, HIDDEN), jnp.float32),
        compiler_params=pltpu.CompilerParams(
            dimension_semantics=("parallel",)),
    )(agg, wvt)


def kernel(x, Wq, keys, Wv, core0, core1, qn_scale, kn_scale, values,
           shuffle_index):
    t = x.shape[0]
    wqt = Wq.T
    kmat = jnp.transpose(keys, (1, 3, 4, 0, 2)).reshape(
        2, KDIM, RANK * HEAD * KEY_NUM)
    qn = qn_scale.reshape(1, KDIM)
    kn = kn_scale.reshape(1, KDIM, 1)
    cflat = jnp.stack([core0, core1], axis=0).reshape(-1)

    w, vi = _stage_a(x, wqt, kmat, qn, kn, cflat)
    agg = _sc_gather_combine(_round_vals(values), shuffle_index,
                             vi.reshape(-1), w.reshape(-1))
    wvt = Wv.T.astype(jnp.bfloat16)
    return _stage_proj(agg, wvt)
